# baseline (speedup 1.0000x reference)
import functools

import jax
import jax.numpy as jnp
from jax import lax
from jax.experimental import pallas as pl
from jax.experimental.pallas import tpu as pltpu
from jax.experimental.pallas import tpu_sc as plsc

FEAT = 32
HID = 128

NC = 2
NS = 16
NW = NC * NS

CB = 80


RB = 128
NSLOT = 4


def _sc_gather_body(nrows_tot, rows_per_w, src_hbm, dst_hbm, hsrc_hbm, hdst_hbm,
                    xs_hbm, xd_hbm, idx_s, idx_d, rows_s, rows_d,
                    isem_s, isem_d, gsem, osem):
    wid = lax.axis_index("s") * NC + lax.axis_index("c")
    row0 = wid * rows_per_w
    rend = jnp.minimum(nrows_tot, row0 + rows_per_w)
    ngroups = (rows_per_w + NSLOT - 1) // NSLOT

    def fire(s, p, r):
        pltpu.async_copy(hsrc_hbm.at[idx_s.at[s, p]], rows_s.at[s], gsem[s])
        pltpu.async_copy(hdst_hbm.at[idx_d.at[s, p]], rows_d.at[s], gsem[s])

    def wait_gathers(s):
        pltpu.make_async_copy(hsrc_hbm.at[pl.ds(0, RB)], rows_s.at[s], gsem[s]).wait()
        pltpu.make_async_copy(hdst_hbm.at[pl.ds(0, RB)], rows_d.at[s], gsem[s]).wait()

    def issue_outs(s, r):
        pltpu.async_copy(rows_s.at[s], xs_hbm.at[pl.ds(r * RB, RB)], osem[s])
        pltpu.async_copy(rows_d.at[s], xd_hbm.at[pl.ds(r * RB, RB)], osem[s])

    def wait_outs(s, r):
        pltpu.make_async_copy(rows_s.at[s], xs_hbm.at[pl.ds(r * RB, RB)], osem[s]).wait()
        pltpu.make_async_copy(rows_d.at[s], xd_hbm.at[pl.ds(r * RB, RB)], osem[s]).wait()

    def issue_idx(s, p, r):
        pltpu.async_copy(src_hbm.at[pl.ds(r * RB, RB)], idx_s.at[s, p], isem_s[s])
        pltpu.async_copy(dst_hbm.at[pl.ds(r * RB, RB)], idx_d.at[s, p], isem_d[s])

    def wait_idx(s, p, r):
        pltpu.make_async_copy(src_hbm.at[pl.ds(r * RB, RB)], idx_s.at[s, p], isem_s[s]).wait()
        pltpu.make_async_copy(dst_hbm.at[pl.ds(r * RB, RB)], idx_d.at[s, p], isem_d[s]).wait()

    for s in range(NSLOT):
        @pl.when(row0 + s < rend)
        def _(s=s):
            issue_idx(s, 0, row0 + s)

    def body(g, carry):
        p = lax.rem(g, 2)
        for s in range(NSLOT):
            r = row0 + g * NSLOT + s

            @pl.when(jnp.logical_and(g > 0, r - NSLOT < rend))
            def _(s=s, r=r):
                wait_gathers(s)
                issue_outs(s, r - NSLOT)
                wait_outs(s, r - NSLOT)

            @pl.when(r < rend)
            def _(s=s, r=r, p=p):
                wait_idx(s, p, r)
                fire(s, p, r)

            @pl.when(r + NSLOT < rend)
            def _(s=s, r=r, p=p):
                issue_idx(s, 1 - p, r + NSLOT)
        return carry

    lax.fori_loop(0, ngroups, body, 0)

    plast = lax.rem(ngroups - 1, 2)
    del plast
    for s in range(NSLOT):
        r = row0 + (ngroups - 1) * NSLOT + s

        @pl.when(r < rend)
        def _(s=s, r=r):
            wait_gathers(s)
            issue_outs(s, r)
            wait_outs(s, r)


def _sc_gather(src_idx, dst_idx, h_src, h_dst):
    n_edges = src_idx.shape[0]
    nrows_tot = n_edges // RB
    rows_per_w = (nrows_tot + NW - 1) // NW
    mesh = plsc.VectorSubcoreMesh(core_axis_name="c", subcore_axis_name="s")
    kern = pl.kernel(
        functools.partial(_sc_gather_body, nrows_tot, rows_per_w),
        mesh=mesh,
        compiler_params=pltpu.CompilerParams(use_tc_tiling_on_sc=False),
        out_type=(
            jax.ShapeDtypeStruct((n_edges, FEAT), jnp.bfloat16),
            jax.ShapeDtypeStruct((n_edges, FEAT), jnp.bfloat16),
        ),
        scratch_types=[
            pltpu.VMEM((NSLOT, 2, RB), jnp.int32),
            pltpu.VMEM((NSLOT, 2, RB), jnp.int32),
            pltpu.VMEM((NSLOT, RB, FEAT), jnp.bfloat16),
            pltpu.VMEM((NSLOT, RB, FEAT), jnp.bfloat16),
            [pltpu.SemaphoreType.DMA] * NSLOT,
            [pltpu.SemaphoreType.DMA] * NSLOT,
            [pltpu.SemaphoreType.DMA] * NSLOT,
            [pltpu.SemaphoreType.DMA] * NSLOT,
        ],
    )
    return kern(src_idx, dst_idx, h_src, h_dst)


PACK = 4


def _tc_body(xs_ref, xd_ref, bdw1_ref, b1t_ref, bdw2_ref, bdo_ref,
             relr_ref, b2_ref, qsel_ref, wsel_ref, out_ref):
    xs = xs_ref[...]
    xd = xd_ref[...]
    cat = jnp.concatenate([xs, xd], axis=1)
    u = jnp.dot(cat, bdw1_ref[...], preferred_element_type=jnp.float32)
    u = u + b1t_ref[...]
    u = jnp.where(u >= 0, u, 0.2 * u)
    glin = jnp.dot(u, bdw2_ref[...], preferred_element_type=jnp.float32) + b2_ref[0, 0]
    g = jax.nn.sigmoid(glin)
    t = jnp.dot(xs * xd * relr_ref[...], bdo_ref[...],
                preferred_element_type=jnp.float32)
    score = g * t
    srep = jnp.dot(score, qsel_ref[...], preferred_element_type=jnp.float32)
    rq = score.shape[0] // 32
    o = jnp.sum(srep.reshape(rq, 32, HID) * wsel_ref[...][None], axis=1)
    out_ref[...] = o[None]


def _tc_mlp(xs_p, xd_p, bdw1, b1t, bdw2, bdo, relr, b2s, qsel, wsel):
    n_rows = xs_p.shape[0]
    rb = 3200
    nb = n_rows // rb
    out = pl.pallas_call(
        _tc_body,
        grid=(nb,),
        in_specs=[
            pl.BlockSpec((rb, HID), lambda e: (e, 0)),
            pl.BlockSpec((rb, HID), lambda e: (e, 0)),
            pl.BlockSpec((2 * HID, PACK * HID), lambda e: (0, 0)),
            pl.BlockSpec((1, PACK * HID), lambda e: (0, 0)),
            pl.BlockSpec((PACK * HID, PACK), lambda e: (0, 0)),
            pl.BlockSpec((HID, PACK), lambda e: (0, 0)),
            pl.BlockSpec((1, HID), lambda e: (0, 0)),
            pl.BlockSpec((1, 1), lambda e: (0, 0)),
            pl.BlockSpec((PACK, HID), lambda e: (0, 0)),
            pl.BlockSpec((32, HID), lambda e: (0, 0)),
        ],
        out_specs=pl.BlockSpec((1, rb // 32, HID), lambda e: (e, 0, 0)),
        out_shape=jax.ShapeDtypeStruct((nb, rb // 32, HID), jnp.float32),
    )(xs_p, xd_p, bdw1, b1t, bdw2, bdo, relr, b2s, qsel, wsel)
    return out.reshape(n_rows * PACK)


def kernel(edge_index, h_src, h_dst, rel_weight, W1, b1, W2, b2):
    n_edges = edge_index.shape[1]
    xs, xd = _sc_gather(edge_index[0], edge_index[1],
                        h_src.astype(jnp.bfloat16), h_dst.astype(jnp.bfloat16))
    xs_p = xs.reshape(n_edges // PACK, PACK * FEAT)
    xd_p = xd.reshape(n_edges // PACK, PACK * FEAT)
    eye4 = jnp.eye(PACK, dtype=jnp.float32)
    b1p = rel_weight @ W1[2 * FEAT:] + b1
    bdw1 = jnp.concatenate(
        [jnp.kron(eye4, W1[:FEAT]), jnp.kron(eye4, W1[FEAT:2 * FEAT])],
        axis=0).astype(jnp.bfloat16)
    b1t = jnp.tile(b1p, PACK).reshape(1, PACK * HID)
    bdw2 = jnp.kron(eye4, W2)
    bdo = jnp.kron(eye4, jnp.ones((FEAT, 1), jnp.float32)).astype(jnp.bfloat16)
    relr = jnp.tile(rel_weight, PACK).reshape(1, HID).astype(jnp.bfloat16)
    b2s = b2.reshape(1, 1)
    lane = jnp.arange(HID, dtype=jnp.int32)
    qsel = (lane[None, :] % PACK == jnp.arange(PACK, dtype=jnp.int32)[:, None]
            ).astype(jnp.float32)
    wsel = (lane[None, :] // PACK == jnp.arange(32, dtype=jnp.int32)[:, None]
            ).astype(jnp.float32)
    return _tc_mlp(xs_p, xd_p, bdw1, b1t, bdw2, bdo, relr, b2s, qsel, wsel)


# speedup vs baseline: 2.3281x; 2.3281x over previous
import functools

import jax
import jax.numpy as jnp
from jax import lax
from jax.experimental import pallas as pl
from jax.experimental.pallas import tpu as pltpu
from jax.experimental.pallas import tpu_sc as plsc

FEAT = 32
HID = 128

NC = 2
NS = 16
NW = NC * NS

CB = 80


RB = 128
NSLOT = 4


def _sc_gather_body(nrows_tot, rows_per_w, src_hbm, dst_hbm, hsrc_hbm, hdst_hbm,
                    xs_hbm, xd_hbm, idx_s, idx_d, rows_s, rows_d,
                    isem_s, isem_d, gsem, osem):
    wid = lax.axis_index("s") * NC + lax.axis_index("c")
    row0 = wid * rows_per_w
    rend = jnp.minimum(nrows_tot, row0 + rows_per_w)
    ngroups = (rows_per_w + NSLOT - 1) // NSLOT

    def fire(s, p, r):
        pltpu.async_copy(hsrc_hbm.at[idx_s.at[s, p]], rows_s.at[s], gsem[s])
        pltpu.async_copy(hdst_hbm.at[idx_d.at[s, p]], rows_d.at[s], gsem[s])

    def wait_gathers(s):
        pltpu.make_async_copy(hsrc_hbm.at[pl.ds(0, RB)], rows_s.at[s], gsem[s]).wait()
        pltpu.make_async_copy(hdst_hbm.at[pl.ds(0, RB)], rows_d.at[s], gsem[s]).wait()

    def issue_outs(s, r):
        pltpu.async_copy(rows_s.at[s], xs_hbm.at[pl.ds(r * RB, RB)], osem[s])
        pltpu.async_copy(rows_d.at[s], xd_hbm.at[pl.ds(r * RB, RB)], osem[s])

    def wait_outs(s, r):
        pltpu.make_async_copy(rows_s.at[s], xs_hbm.at[pl.ds(r * RB, RB)], osem[s]).wait()
        pltpu.make_async_copy(rows_d.at[s], xd_hbm.at[pl.ds(r * RB, RB)], osem[s]).wait()

    def issue_idx(s, p, r):
        pltpu.async_copy(src_hbm.at[pl.ds(r * RB, RB)], idx_s.at[s, p], isem_s[s])
        pltpu.async_copy(dst_hbm.at[pl.ds(r * RB, RB)], idx_d.at[s, p], isem_d[s])

    def wait_idx(s, p, r):
        pltpu.make_async_copy(src_hbm.at[pl.ds(r * RB, RB)], idx_s.at[s, p], isem_s[s]).wait()
        pltpu.make_async_copy(dst_hbm.at[pl.ds(r * RB, RB)], idx_d.at[s, p], isem_d[s]).wait()

    for s in range(NSLOT):
        @pl.when(row0 + s < rend)
        def _(s=s):
            issue_idx(s, 0, row0 + s)

    def body(g, carry):
        p = lax.rem(g, 2)
        for s in range(NSLOT):
            r = row0 + g * NSLOT + s

            @pl.when(jnp.logical_and(g > 0, r - NSLOT < rend))
            def _(s=s, r=r):
                wait_gathers(s)
                issue_outs(s, r - NSLOT)
                wait_outs(s, r - NSLOT)

            @pl.when(r < rend)
            def _(s=s, r=r, p=p):
                wait_idx(s, p, r)
                fire(s, p, r)

            @pl.when(r + NSLOT < rend)
            def _(s=s, r=r, p=p):
                issue_idx(s, 1 - p, r + NSLOT)
        return carry

    lax.fori_loop(0, ngroups, body, 0)

    plast = lax.rem(ngroups - 1, 2)
    del plast
    for s in range(NSLOT):
        r = row0 + (ngroups - 1) * NSLOT + s

        @pl.when(r < rend)
        def _(s=s, r=r):
            wait_gathers(s)
            issue_outs(s, r)
            wait_outs(s, r)


def _sc_gather(src_idx, dst_idx, h_src, h_dst):
    n_edges = src_idx.shape[0]
    nrows_tot = n_edges // RB
    rows_per_w = (nrows_tot + NW - 1) // NW
    mesh = plsc.VectorSubcoreMesh(core_axis_name="c", subcore_axis_name="s")
    kern = pl.kernel(
        functools.partial(_sc_gather_body, nrows_tot, rows_per_w),
        mesh=mesh,
        compiler_params=pltpu.CompilerParams(use_tc_tiling_on_sc=False),
        out_type=(
            jax.ShapeDtypeStruct((n_edges, FEAT), jnp.float32),
            jax.ShapeDtypeStruct((n_edges, FEAT), jnp.float32),
        ),
        scratch_types=[
            pltpu.VMEM((NSLOT, 2, RB), jnp.int32),
            pltpu.VMEM((NSLOT, 2, RB), jnp.int32),
            pltpu.VMEM((NSLOT, RB, FEAT), jnp.float32),
            pltpu.VMEM((NSLOT, RB, FEAT), jnp.float32),
            [pltpu.SemaphoreType.DMA] * NSLOT,
            [pltpu.SemaphoreType.DMA] * NSLOT,
            [pltpu.SemaphoreType.DMA] * NSLOT,
            [pltpu.SemaphoreType.DMA] * NSLOT,
        ],
    )
    return kern(src_idx, dst_idx, h_src, h_dst)


PACK = 4


def _tc_body(xs_ref, xd_ref, bdw1_ref, b1t_ref, bdw2_ref, bdo_ref,
             relr_ref, b2_ref, qsel_ref, wsel_ref, out_ref):
    xs = xs_ref[...]
    xd = xd_ref[...]
    cat = jnp.concatenate([xs, xd], axis=1).astype(jnp.bfloat16)
    u = jnp.dot(cat, bdw1_ref[...], preferred_element_type=jnp.float32)
    u = u + b1t_ref[...]
    u = jnp.where(u >= 0, u, 0.2 * u)
    glin = jnp.dot(u, bdw2_ref[...], preferred_element_type=jnp.float32) + b2_ref[0, 0]
    g = jax.nn.sigmoid(glin)
    t = jnp.dot(xs * xd * relr_ref[...], bdo_ref[...],
                preferred_element_type=jnp.float32)
    score = g * t
    srep = jnp.dot(score, qsel_ref[...], preferred_element_type=jnp.float32)
    rq = score.shape[0] // 32
    o = jnp.sum(srep.reshape(rq, 32, HID) * wsel_ref[...][None], axis=1)
    out_ref[...] = o[None]


def _pick_rb(n_rows):
    rb = min(n_rows, 4096) // 32 * 32
    while n_rows % rb:
        rb -= 32
    return rb


def _tc_mlp(xs_p, xd_p, bdw1, b1t, bdw2, bdo, relr, b2s, qsel, wsel):
    n_rows = xs_p.shape[0]
    rb = _pick_rb(n_rows)
    nb = n_rows // rb
    out = pl.pallas_call(
        _tc_body,
        grid=(nb,),
        in_specs=[
            pl.BlockSpec((rb, HID), lambda e: (e, 0)),
            pl.BlockSpec((rb, HID), lambda e: (e, 0)),
            pl.BlockSpec((2 * HID, PACK * HID), lambda e: (0, 0)),
            pl.BlockSpec((1, PACK * HID), lambda e: (0, 0)),
            pl.BlockSpec((PACK * HID, PACK), lambda e: (0, 0)),
            pl.BlockSpec((HID, PACK), lambda e: (0, 0)),
            pl.BlockSpec((1, HID), lambda e: (0, 0)),
            pl.BlockSpec((1, 1), lambda e: (0, 0)),
            pl.BlockSpec((PACK, HID), lambda e: (0, 0)),
            pl.BlockSpec((32, HID), lambda e: (0, 0)),
        ],
        out_specs=pl.BlockSpec((1, rb // 32, HID), lambda e: (e, 0, 0)),
        out_shape=jax.ShapeDtypeStruct((nb, rb // 32, HID), jnp.float32),
    )(xs_p, xd_p, bdw1, b1t, bdw2, bdo, relr, b2s, qsel, wsel)
    return out.reshape(n_rows * PACK)


def kernel(edge_index, h_src, h_dst, rel_weight, W1, b1, W2, b2):
    n_edges = edge_index.shape[1]
    eye4 = jnp.eye(PACK, dtype=jnp.float32)
    b1p = rel_weight @ W1[2 * FEAT:] + b1
    bdw1 = jnp.concatenate(
        [jnp.kron(eye4, W1[:FEAT]), jnp.kron(eye4, W1[FEAT:2 * FEAT])],
        axis=0).astype(jnp.bfloat16)
    b1t = jnp.tile(b1p, PACK).reshape(1, PACK * HID)
    bdw2 = jnp.kron(eye4, W2)
    bdo = jnp.kron(eye4, jnp.ones((FEAT, 1), jnp.float32))
    relr = jnp.tile(rel_weight, PACK).reshape(1, HID)
    b2s = b2.reshape(1, 1)
    lane = jnp.arange(HID, dtype=jnp.int32)
    qsel = (lane[None, :] % PACK == jnp.arange(PACK, dtype=jnp.int32)[:, None]
            ).astype(jnp.float32)
    wsel = (lane[None, :] // PACK == jnp.arange(32, dtype=jnp.int32)[:, None]
            ).astype(jnp.float32)
    nch = 2
    ec = n_edges // nch
    scores = []
    for i in range(nch):
        xs, xd = _sc_gather(edge_index[0, i * ec:(i + 1) * ec],
                            edge_index[1, i * ec:(i + 1) * ec], h_src, h_dst)
        xs_p = xs.reshape(ec // PACK, PACK * FEAT)
        xd_p = xd.reshape(ec // PACK, PACK * FEAT)
        scores.append(_tc_mlp(xs_p, xd_p, bdw1, b1t, bdw2, bdo, relr, b2s,
                              qsel, wsel))
    return jnp.concatenate(scores)


# speedup vs baseline: 2.4132x; 1.0365x over previous
import functools

import jax
import jax.numpy as jnp
from jax import lax
from jax.experimental import pallas as pl
from jax.experimental.pallas import tpu as pltpu
from jax.experimental.pallas import tpu_sc as plsc

FEAT = 32
HID = 128

NC = 2
NS = 16
NW = NC * NS

CB = 80


RB = 128
NSLOT = 4


def _sc_gather_body(nrows_tot, rows_per_w, src_hbm, dst_hbm, hsrc_hbm, hdst_hbm,
                    xs_hbm, xd_hbm, idx_s, idx_d, rows_s, rows_d,
                    isem_s, isem_d, gsem, osem):
    wid = lax.axis_index("s") * NC + lax.axis_index("c")
    row0 = wid * rows_per_w
    rend = jnp.minimum(nrows_tot, row0 + rows_per_w)
    ngroups = (rows_per_w + NSLOT - 1) // NSLOT

    def fire(s, p, r):
        pltpu.async_copy(hsrc_hbm.at[idx_s.at[s, p]], rows_s.at[s], gsem[s])
        pltpu.async_copy(hdst_hbm.at[idx_d.at[s, p]], rows_d.at[s], gsem[s])

    def wait_gathers(s):
        pltpu.make_async_copy(hsrc_hbm.at[pl.ds(0, RB)], rows_s.at[s], gsem[s]).wait()
        pltpu.make_async_copy(hdst_hbm.at[pl.ds(0, RB)], rows_d.at[s], gsem[s]).wait()

    def issue_outs(s, r):
        pltpu.async_copy(rows_s.at[s], xs_hbm.at[pl.ds(r * RB, RB)], osem[s])
        pltpu.async_copy(rows_d.at[s], xd_hbm.at[pl.ds(r * RB, RB)], osem[s])

    def wait_outs(s, r):
        pltpu.make_async_copy(rows_s.at[s], xs_hbm.at[pl.ds(r * RB, RB)], osem[s]).wait()
        pltpu.make_async_copy(rows_d.at[s], xd_hbm.at[pl.ds(r * RB, RB)], osem[s]).wait()

    def issue_idx(s, p, r):
        pltpu.async_copy(src_hbm.at[pl.ds(r * RB, RB)], idx_s.at[s, p], isem_s[s])
        pltpu.async_copy(dst_hbm.at[pl.ds(r * RB, RB)], idx_d.at[s, p], isem_d[s])

    def wait_idx(s, p, r):
        pltpu.make_async_copy(src_hbm.at[pl.ds(r * RB, RB)], idx_s.at[s, p], isem_s[s]).wait()
        pltpu.make_async_copy(dst_hbm.at[pl.ds(r * RB, RB)], idx_d.at[s, p], isem_d[s]).wait()

    for s in range(NSLOT):
        @pl.when(row0 + s < rend)
        def _(s=s):
            issue_idx(s, 0, row0 + s)

    def body(g, carry):
        p = lax.rem(g, 2)
        for s in range(NSLOT):
            r = row0 + g * NSLOT + s

            @pl.when(jnp.logical_and(g > 0, r - NSLOT < rend))
            def _(s=s, r=r):
                wait_gathers(s)
                issue_outs(s, r - NSLOT)
                wait_outs(s, r - NSLOT)

            @pl.when(r < rend)
            def _(s=s, r=r, p=p):
                wait_idx(s, p, r)
                fire(s, p, r)

            @pl.when(r + NSLOT < rend)
            def _(s=s, r=r, p=p):
                issue_idx(s, 1 - p, r + NSLOT)
        return carry

    lax.fori_loop(0, ngroups, body, 0)

    plast = lax.rem(ngroups - 1, 2)
    del plast
    for s in range(NSLOT):
        r = row0 + (ngroups - 1) * NSLOT + s

        @pl.when(r < rend)
        def _(s=s, r=r):
            wait_gathers(s)
            issue_outs(s, r)
            wait_outs(s, r)


def _sc_gather(src_idx, dst_idx, h_src, h_dst):
    n_edges = src_idx.shape[0]
    nrows_tot = n_edges // RB
    rows_per_w = (nrows_tot + NW - 1) // NW
    mesh = plsc.VectorSubcoreMesh(core_axis_name="c", subcore_axis_name="s")
    kern = pl.kernel(
        functools.partial(_sc_gather_body, nrows_tot, rows_per_w),
        mesh=mesh,
        compiler_params=pltpu.CompilerParams(use_tc_tiling_on_sc=False),
        out_type=(
            jax.ShapeDtypeStruct((n_edges, FEAT), jnp.float32),
            jax.ShapeDtypeStruct((n_edges, FEAT), jnp.float32),
        ),
        scratch_types=[
            pltpu.VMEM((NSLOT, 2, RB), jnp.int32),
            pltpu.VMEM((NSLOT, 2, RB), jnp.int32),
            pltpu.VMEM((NSLOT, RB, FEAT), jnp.float32),
            pltpu.VMEM((NSLOT, RB, FEAT), jnp.float32),
            [pltpu.SemaphoreType.DMA] * NSLOT,
            [pltpu.SemaphoreType.DMA] * NSLOT,
            [pltpu.SemaphoreType.DMA] * NSLOT,
            [pltpu.SemaphoreType.DMA] * NSLOT,
        ],
    )
    return kern(src_idx, dst_idx, h_src, h_dst)


PACK = 4


def _tc_body(xs_ref, xd_ref, bdw1_ref, b1t_ref, bdw2_ref, bdo_ref,
             relr_ref, b2_ref, qsel_ref, wsel_ref, out_ref):
    xs = xs_ref[...]
    xd = xd_ref[...]
    cat = jnp.concatenate([xs, xd], axis=1).astype(jnp.bfloat16)
    u = jnp.dot(cat, bdw1_ref[...], preferred_element_type=jnp.float32)
    u = u + b1t_ref[...]
    u = jnp.where(u >= 0, u, 0.2 * u)
    glin = jnp.dot(u, bdw2_ref[...], preferred_element_type=jnp.float32) + b2_ref[0, 0]
    g = jax.nn.sigmoid(glin)
    t = jnp.dot(xs * xd * relr_ref[...], bdo_ref[...],
                preferred_element_type=jnp.float32)
    score = g * t
    srep = jnp.dot(score, qsel_ref[...], preferred_element_type=jnp.float32)
    rq = score.shape[0] // 32
    o = jnp.sum(srep.reshape(rq, 32, HID) * wsel_ref[...][None], axis=1)
    out_ref[...] = o[None]


def _pick_rb(n_rows):
    rb = min(n_rows, 4096) // 32 * 32
    while n_rows % rb:
        rb -= 32
    return rb


def _tc_mlp(xs_p, xd_p, bdw1, b1t, bdw2, bdo, relr, b2s, qsel, wsel):
    n_rows = xs_p.shape[0]
    rb = _pick_rb(n_rows)
    nb = n_rows // rb
    out = pl.pallas_call(
        _tc_body,
        grid=(nb,),
        in_specs=[
            pl.BlockSpec((rb, HID), lambda e: (e, 0)),
            pl.BlockSpec((rb, HID), lambda e: (e, 0)),
            pl.BlockSpec((2 * HID, PACK * HID), lambda e: (0, 0)),
            pl.BlockSpec((1, PACK * HID), lambda e: (0, 0)),
            pl.BlockSpec((PACK * HID, PACK), lambda e: (0, 0)),
            pl.BlockSpec((HID, PACK), lambda e: (0, 0)),
            pl.BlockSpec((1, HID), lambda e: (0, 0)),
            pl.BlockSpec((1, 1), lambda e: (0, 0)),
            pl.BlockSpec((PACK, HID), lambda e: (0, 0)),
            pl.BlockSpec((32, HID), lambda e: (0, 0)),
        ],
        out_specs=pl.BlockSpec((1, rb // 32, HID), lambda e: (e, 0, 0)),
        out_shape=jax.ShapeDtypeStruct((nb, rb // 32, HID), jnp.float32),
    )(xs_p, xd_p, bdw1, b1t, bdw2, bdo, relr, b2s, qsel, wsel)
    return out.reshape(n_rows * PACK)


def kernel(edge_index, h_src, h_dst, rel_weight, W1, b1, W2, b2):
    n_edges = edge_index.shape[1]
    eye4 = jnp.eye(PACK, dtype=jnp.float32)
    b1p = rel_weight @ W1[2 * FEAT:] + b1
    bdw1 = jnp.concatenate(
        [jnp.kron(eye4, W1[:FEAT]), jnp.kron(eye4, W1[FEAT:2 * FEAT])],
        axis=0).astype(jnp.bfloat16)
    b1t = jnp.tile(b1p, PACK).reshape(1, PACK * HID)
    bdw2 = jnp.kron(eye4, W2)
    bdo = jnp.kron(eye4, jnp.ones((FEAT, 1), jnp.float32))
    relr = jnp.tile(rel_weight, PACK).reshape(1, HID)
    b2s = b2.reshape(1, 1)
    lane = jnp.arange(HID, dtype=jnp.int32)
    qsel = (lane[None, :] % PACK == jnp.arange(PACK, dtype=jnp.int32)[:, None]
            ).astype(jnp.float32)
    wsel = (lane[None, :] // PACK == jnp.arange(32, dtype=jnp.int32)[:, None]
            ).astype(jnp.float32)
    nch = 4
    ec = n_edges // nch
    scores = []
    for i in range(nch):
        xs, xd = _sc_gather(edge_index[0, i * ec:(i + 1) * ec],
                            edge_index[1, i * ec:(i + 1) * ec], h_src, h_dst)
        xs_p = xs.reshape(ec // PACK, PACK * FEAT)
        xd_p = xd.reshape(ec // PACK, PACK * FEAT)
        scores.append(_tc_mlp(xs_p, xd_p, bdw1, b1t, bdw2, bdo, relr, b2s,
                              qsel, wsel))
    return jnp.concatenate(scores)
